# baseline (speedup 1.0000x reference)
import functools

import jax
import jax.numpy as jnp
from jax import lax
from jax.experimental import pallas as pl
from jax.experimental.pallas import tpu as pltpu
from jax.experimental.pallas import tpu_sc as plsc

BATCH = 16384
D = 32
L = 16

N_U = 1000000
NTILE_U = (N_U + 127) // 128
N_FULL = N_U // 128
NBUF = 32


def _copy_body(ut3_hbm, out_hbm, bufa, bufb,
               sem_ia, sem_ib, sem_oa, sem_ob, *, nc):
    wid = lax.axis_index("s") * nc + lax.axis_index("c")
    nw = 32

    n_steps = (4 * N_FULL + nw - 1 - wid) // nw

    def src_dst(k):
        t = wid + k * nw
        td = t // N_FULL
        tr = t - td * N_FULL
        row0 = pl.multiple_of((td * NTILE_U + tr) * 8, 8)
        col0 = pl.multiple_of(tr * 128, 128)
        return td, col0, row0

    def fire_in(k, buf, sem, b):
        td, col0, _ = src_dst(k)
        pltpu.async_copy(ut3_hbm.at[td, :, pl.ds(col0, 128)],
                         buf.at[pl.ds(b * 8, 8), :], sem)

    def fire_out(k, buf, sem, b):
        _, _, row0 = src_dst(k)
        pltpu.async_copy(buf.at[pl.ds(b * 8, 8), :],
                         out_hbm.at[pl.ds(row0, 8), :], sem)

    def drain_one(sem):
        pltpu.make_async_copy(
            ut3_hbm.at[0, :, pl.ds(0, 128)], bufa.at[pl.ds(0, 8), :], sem
        ).wait()

    bank_of = (
        (bufa, sem_ia, sem_oa), (bufb, sem_ib, sem_ob))
    n_outer_val = (n_steps + NBUF - 1) // NBUF

    def outer(k2, _):
        for parity in range(2):
            @pl.when(k2 % 2 == parity)
            def _(parity=parity):
                buf, sem_i, sem_o = bank_of[parity]
                for b in range(NBUF):
                    s = (k2 - 2) * NBUF + b
                    @pl.when(jnp.logical_and(k2 >= 2, s < n_steps))
                    def _(b=b, s=s):
                        drain_one(sem_o)
                for b in range(NBUF):
                    s = k2 * NBUF + b
                    @pl.when(s < n_steps)
                    def _(b=b, s=s):
                        fire_in(s, buf, sem_i, b)
                pbuf, psem_i, psem_o = bank_of[1 - parity]
                for b in range(NBUF):
                    s = (k2 - 1) * NBUF + b
                    @pl.when(jnp.logical_and(k2 >= 1, s < n_steps))
                    def _(b=b, s=s):
                        drain_one(psem_i)
                for b in range(NBUF):
                    s = (k2 - 1) * NBUF + b
                    @pl.when(jnp.logical_and(k2 >= 1, s < n_steps))
                    def _(b=b, s=s):
                        fire_out(s, pbuf, psem_o, b)
        return _

    lax.fori_loop(0, n_outer_val + 3, outer, None)


def _gather_body(u_hbm, i_hbm, j_hbm, ulin_hbm, v_hbm, utail_hbm, out_hbm,
                 idx_u, idx_i, idx_j, widx, urows, irows, jrows, utail_v,
                 outv, sem_u, sem_v, *, b_per_w, nc):
    wid = lax.axis_index("s") * nc + lax.axis_index("c")
    base = wid * b_per_w
    nrow = b_per_w // 128
    row0 = wid * nrow

    pltpu.sync_copy(u_hbm.at[pl.ds(row0, nrow)], idx_u)
    pltpu.sync_copy(i_hbm.at[pl.ds(row0, nrow)], idx_i)
    pltpu.sync_copy(j_hbm.at[pl.ds(row0, nrow)], idx_j)
    pltpu.sync_copy(utail_hbm, utail_v)

    hv = []
    for c in range(nrow):
        sl = pl.ds(c * 128, 128)
        hv.append(pltpu.async_copy(v_hbm.at[idx_i.at[c]], irows.at[sl], sem_v))
        hv.append(pltpu.async_copy(v_hbm.at[idx_j.at[c]], jrows.at[sl], sem_v))

    lane_iota = lax.iota(jnp.int32, L)

    def widx_step(g, _):
        pos = g * L + lane_iota
        r = plsc.load_gather(idx_u, [pos >> 7, pos & 127])
        b0 = ((r >> 7) << 10) + (r & 127)
        for d in range(D):
            w = b0 + ((d // 8) * NTILE_U * 1024 + (d % 8) * 128)
            widx[d, pl.ds(g * L, L)] = w
        return _

    lax.fori_loop(0, b_per_w // L, widx_step, None)

    ngrp = b_per_w // 128
    hu = []
    for d in range(D):
        for c in range(ngrp):
            off = c * 128
            hu.append(pltpu.async_copy(
                ulin_hbm.at[widx.at[d, pl.ds(off, 128)]],
                urows.at[d, pl.ds(off, 128)], sem_u))
    for h in hv:
        h.wait()
    for h in hu:
        h.wait()

    def body(g, _):
        rows16 = g * L + lane_iota
        pos = g * L + lane_iota
        r = plsc.load_gather(idx_u, [pos >> 7, pos & 127])
        is_tail = r >= N_FULL * 128
        tr = jnp.maximum(r - N_FULL * 128, 0)
        acc = None
        for d in range(D):
            dcol = jnp.full((L,), d, jnp.int32)
            diff = (plsc.load_gather(irows, [rows16, dcol])
                    - plsc.load_gather(jrows, [rows16, dcol]))
            uv = urows[d, pl.ds(g * L, L)]
            tv = plsc.load_gather(utail_v, [tr, dcol])
            term = jnp.where(is_tail, tv, uv) * diff
            acc = term if acc is None else acc + term
        outv[pl.ds(g * L, L)] = 1.0 / (1.0 + jnp.exp(-acc))
        return _

    lax.fori_loop(0, b_per_w // L, body, None)

    pltpu.sync_copy(outv, out_hbm.at[pl.ds(base, b_per_w)])


def kernel(u, i, j, U, V):
    try:
        info = plsc.get_sparse_core_info()
        nc, ns = info.num_cores, info.num_subcores
    except ValueError:
        nc, ns = 2, 16
    nw = nc * ns
    b_per_w = BATCH // nw

    mesh = plsc.VectorSubcoreMesh(core_axis_name="c", subcore_axis_name="s")

    ut3 = U.T.reshape(4, 8, N_U)
    copy_k = functools.partial(
        pl.kernel,
        mesh=mesh,
        out_type=jax.ShapeDtypeStruct((4 * NTILE_U * 8, 128), jnp.float32),
        compiler_params=pltpu.CompilerParams(
            needs_layout_passes=False, use_tc_tiling_on_sc=True
        ),
        scratch_types=[
            pltpu.VMEM((8 * NBUF, 128), jnp.float32),
            pltpu.VMEM((8 * NBUF, 128), jnp.float32),
            pltpu.SemaphoreType.DMA,
            pltpu.SemaphoreType.DMA,
            pltpu.SemaphoreType.DMA,
            pltpu.SemaphoreType.DMA,
        ],
    )(functools.partial(_copy_body, nc=nc))
    ulin = copy_k(ut3).reshape(4 * NTILE_U * 8 * 128)

    u2 = u.astype(jnp.int32).reshape(BATCH // 128, 128)
    i2 = i.astype(jnp.int32).reshape(BATCH // 128, 128)
    j2 = j.astype(jnp.int32).reshape(BATCH // 128, 128)

    gather_k = functools.partial(
        pl.kernel,
        mesh=mesh,
        out_type=jax.ShapeDtypeStruct((BATCH,), jnp.float32),
        compiler_params=pltpu.CompilerParams(
            needs_layout_passes=False, use_tc_tiling_on_sc=False
        ),
        scratch_types=[
            pltpu.VMEM((BATCH // 128 // nw, 128), jnp.int32),
            pltpu.VMEM((BATCH // 128 // nw, 128), jnp.int32),
            pltpu.VMEM((BATCH // 128 // nw, 128), jnp.int32),
            pltpu.VMEM((D, b_per_w), jnp.int32),
            pltpu.VMEM((D, b_per_w), jnp.float32),
            pltpu.VMEM((b_per_w, D), jnp.float32),
            pltpu.VMEM((b_per_w, D), jnp.float32),
            pltpu.VMEM((N_U - N_FULL * 128, D), jnp.float32),
            pltpu.VMEM((b_per_w,), jnp.float32),
            pltpu.SemaphoreType.DMA,
            pltpu.SemaphoreType.DMA,
        ],
    )(functools.partial(_gather_body, b_per_w=b_per_w, nc=nc))
    utail = U[N_FULL * 128:, :]
    return gather_k(u2, i2, j2, ulin, V, utail)


# speedup vs baseline: 1.0178x; 1.0178x over previous
import functools

import jax
import jax.numpy as jnp
from jax import lax
from jax.experimental import pallas as pl
from jax.experimental.pallas import tpu as pltpu
from jax.experimental.pallas import tpu_sc as plsc

BATCH = 16384
D = 32
L = 16

N_U = 1000000
NTILE_U = (N_U + 127) // 128
N_FULL = N_U // 128
NBUF = 16


def _copy_body(ut3_hbm, out_hbm, bufa, bufb,
               sem_ia, sem_ib, sem_oa, sem_ob, *, nc):
    wid = lax.axis_index("s") * nc + lax.axis_index("c")
    nw = 32

    n_steps = (4 * N_FULL + nw - 1 - wid) // nw

    def src_dst(k):
        t = wid + k * nw
        td = t // N_FULL
        tr = t - td * N_FULL
        row0 = pl.multiple_of((td * NTILE_U + tr) * 8, 8)
        col0 = pl.multiple_of(tr * 128, 128)
        return td, col0, row0

    def fire_in(k, buf, sem, b):
        td, col0, _ = src_dst(k)
        pltpu.async_copy(ut3_hbm.at[td, :, pl.ds(col0, 128)],
                         buf.at[pl.ds(b * 8, 8), :], sem)

    def fire_out(k, buf, sem, b):
        _, _, row0 = src_dst(k)
        pltpu.async_copy(buf.at[pl.ds(b * 8, 8), :],
                         out_hbm.at[pl.ds(row0, 8), :], sem)

    def drain_one(sem):
        pltpu.make_async_copy(
            ut3_hbm.at[0, :, pl.ds(0, 128)], bufa.at[pl.ds(0, 8), :], sem
        ).wait()

    bank_of = (
        (bufa, sem_ia, sem_oa), (bufb, sem_ib, sem_ob))
    n_outer_val = (n_steps + NBUF - 1) // NBUF

    def outer(k2, _):
        for parity in range(2):
            @pl.when(k2 % 2 == parity)
            def _(parity=parity):
                buf, sem_i, sem_o = bank_of[parity]
                for b in range(NBUF):
                    s = (k2 - 2) * NBUF + b
                    @pl.when(jnp.logical_and(k2 >= 2, s < n_steps))
                    def _(b=b, s=s):
                        drain_one(sem_o)
                for b in range(NBUF):
                    s = k2 * NBUF + b
                    @pl.when(s < n_steps)
                    def _(b=b, s=s):
                        fire_in(s, buf, sem_i, b)
                pbuf, psem_i, psem_o = bank_of[1 - parity]
                for b in range(NBUF):
                    s = (k2 - 1) * NBUF + b
                    @pl.when(jnp.logical_and(k2 >= 1, s < n_steps))
                    def _(b=b, s=s):
                        drain_one(psem_i)
                for b in range(NBUF):
                    s = (k2 - 1) * NBUF + b
                    @pl.when(jnp.logical_and(k2 >= 1, s < n_steps))
                    def _(b=b, s=s):
                        fire_out(s, pbuf, psem_o, b)
        return _

    lax.fori_loop(0, n_outer_val + 3, outer, None)


def _gather_body(u_hbm, i_hbm, j_hbm, ulin_hbm, v_hbm, utail_hbm, out_hbm,
                 idx_u, idx_i, idx_j, widx, urows, irows, jrows, utail_v,
                 outv, sem_u, sem_v, *, b_per_w, nc):
    wid = lax.axis_index("s") * nc + lax.axis_index("c")
    base = wid * b_per_w
    nrow = b_per_w // 128
    row0 = wid * nrow

    pltpu.sync_copy(u_hbm.at[pl.ds(row0, nrow)], idx_u)
    pltpu.sync_copy(i_hbm.at[pl.ds(row0, nrow)], idx_i)
    pltpu.sync_copy(j_hbm.at[pl.ds(row0, nrow)], idx_j)
    pltpu.sync_copy(utail_hbm, utail_v)

    hv = []
    for c in range(nrow):
        sl = pl.ds(c * 128, 128)
        hv.append(pltpu.async_copy(v_hbm.at[idx_i.at[c]], irows.at[sl], sem_v))
        hv.append(pltpu.async_copy(v_hbm.at[idx_j.at[c]], jrows.at[sl], sem_v))

    lane_iota = lax.iota(jnp.int32, L)

    def widx_step(g, _):
        pos = g * L + lane_iota
        r = plsc.load_gather(idx_u, [pos >> 7, pos & 127])
        b0 = ((r >> 7) << 10) + (r & 127)
        for d in range(D):
            w = b0 + ((d // 8) * NTILE_U * 1024 + (d % 8) * 128)
            widx[d, pl.ds(g * L, L)] = w
        return _

    lax.fori_loop(0, b_per_w // L, widx_step, None)

    ngrp = b_per_w // 128
    hu = []
    for d in range(D):
        for c in range(ngrp):
            off = c * 128
            hu.append(pltpu.async_copy(
                ulin_hbm.at[widx.at[d, pl.ds(off, 128)]],
                urows.at[d, pl.ds(off, 128)], sem_u))
    for h in hv:
        h.wait()
    for h in hu:
        h.wait()

    def body(g, _):
        rows16 = g * L + lane_iota
        pos = g * L + lane_iota
        r = plsc.load_gather(idx_u, [pos >> 7, pos & 127])
        is_tail = r >= N_FULL * 128
        tr = jnp.maximum(r - N_FULL * 128, 0)
        acc = None
        for d in range(D):
            dcol = jnp.full((L,), d, jnp.int32)
            diff = (plsc.load_gather(irows, [rows16, dcol])
                    - plsc.load_gather(jrows, [rows16, dcol]))
            uv = urows[d, pl.ds(g * L, L)]
            tv = plsc.load_gather(utail_v, [tr, dcol])
            term = jnp.where(is_tail, tv, uv) * diff
            acc = term if acc is None else acc + term
        outv[pl.ds(g * L, L)] = 1.0 / (1.0 + jnp.exp(-acc))
        return _

    lax.fori_loop(0, b_per_w // L, body, None)

    pltpu.sync_copy(outv, out_hbm.at[pl.ds(base, b_per_w)])


def kernel(u, i, j, U, V):
    try:
        info = plsc.get_sparse_core_info()
        nc, ns = info.num_cores, info.num_subcores
    except ValueError:
        nc, ns = 2, 16
    nw = nc * ns
    b_per_w = BATCH // nw

    mesh = plsc.VectorSubcoreMesh(core_axis_name="c", subcore_axis_name="s")

    ut3 = U.T.reshape(4, 8, N_U)
    copy_k = functools.partial(
        pl.kernel,
        mesh=mesh,
        out_type=jax.ShapeDtypeStruct((4 * NTILE_U * 8, 128), jnp.float32),
        compiler_params=pltpu.CompilerParams(
            needs_layout_passes=False, use_tc_tiling_on_sc=True
        ),
        scratch_types=[
            pltpu.VMEM((8 * NBUF, 128), jnp.float32),
            pltpu.VMEM((8 * NBUF, 128), jnp.float32),
            pltpu.SemaphoreType.DMA,
            pltpu.SemaphoreType.DMA,
            pltpu.SemaphoreType.DMA,
            pltpu.SemaphoreType.DMA,
        ],
    )(functools.partial(_copy_body, nc=nc))
    ulin = copy_k(ut3).reshape(4 * NTILE_U * 8 * 128)

    u2 = u.astype(jnp.int32).reshape(BATCH // 128, 128)
    i2 = i.astype(jnp.int32).reshape(BATCH // 128, 128)
    j2 = j.astype(jnp.int32).reshape(BATCH // 128, 128)

    gather_k = functools.partial(
        pl.kernel,
        mesh=mesh,
        out_type=jax.ShapeDtypeStruct((BATCH,), jnp.float32),
        compiler_params=pltpu.CompilerParams(
            needs_layout_passes=False, use_tc_tiling_on_sc=False
        ),
        scratch_types=[
            pltpu.VMEM((BATCH // 128 // nw, 128), jnp.int32),
            pltpu.VMEM((BATCH // 128 // nw, 128), jnp.int32),
            pltpu.VMEM((BATCH // 128 // nw, 128), jnp.int32),
            pltpu.VMEM((D, b_per_w), jnp.int32),
            pltpu.VMEM((D, b_per_w), jnp.float32),
            pltpu.VMEM((b_per_w, D), jnp.float32),
            pltpu.VMEM((b_per_w, D), jnp.float32),
            pltpu.VMEM((N_U - N_FULL * 128, D), jnp.float32),
            pltpu.VMEM((b_per_w,), jnp.float32),
            pltpu.SemaphoreType.DMA,
            pltpu.SemaphoreType.DMA,
        ],
    )(functools.partial(_gather_body, b_per_w=b_per_w, nc=nc))
    utail = U[N_FULL * 128:, :]
    return gather_k(u2, i2, j2, ulin, V, utail)


# speedup vs baseline: 1.0213x; 1.0034x over previous
import functools

import jax
import jax.numpy as jnp
from jax import lax
from jax.experimental import pallas as pl
from jax.experimental.pallas import tpu as pltpu
from jax.experimental.pallas import tpu_sc as plsc

BATCH = 16384
D = 32
L = 16

N_U = 1000000
NTILE_U = (N_U + 127) // 128
N_FULL = N_U // 128
NBUF = 8
KTR = 4


def _copy_body(ut3_hbm, out_hbm, bufa, bufb,
               sem_ia, sem_ib, sem_oa, sem_ob, *, nc):
    wid = lax.axis_index("s") * nc + lax.axis_index("c")
    nw = 32

    n_grp_td = N_FULL // KTR
    n_steps = (4 * n_grp_td + nw - 1 - wid) // nw

    def src_dst(k):
        g = wid + k * nw
        td = g // n_grp_td
        tr0 = (g - td * n_grp_td) * KTR
        row0 = pl.multiple_of((td * NTILE_U + tr0) * 8, 8)
        col0 = pl.multiple_of(tr0 * 128, 128)
        return td, col0, row0

    def fire_in(k, buf, sem, b):
        td, col0, _ = src_dst(k)
        pltpu.async_copy(ut3_hbm.at[td, :, pl.ds(col0, 128 * KTR)],
                         buf.at[pl.ds(b * 8, 8), :], sem)

    def fire_out(k, buf, sem, b):
        _, _, row0 = src_dst(k)
        for q in range(KTR):
            pltpu.async_copy(
                buf.at[pl.ds(b * 8, 8), pl.ds(q * 128, 128)],
                out_hbm.at[pl.ds(row0 + q * 8, 8), :], sem)

    def drain_in_one(sem):
        pltpu.make_async_copy(
            ut3_hbm.at[0, :, pl.ds(0, 128 * KTR)],
            bufa.at[pl.ds(0, 8), :], sem).wait()

    def drain_one(sem):
        for q in range(KTR):
            pltpu.make_async_copy(
                ut3_hbm.at[0, :, pl.ds(0, 128)],
                bufa.at[pl.ds(0, 8), pl.ds(0, 128)], sem).wait()

    bank_of = (
        (bufa, sem_ia, sem_oa), (bufb, sem_ib, sem_ob))
    n_outer_val = (n_steps + NBUF - 1) // NBUF

    def outer(k2, _):
        for parity in range(2):
            @pl.when(k2 % 2 == parity)
            def _(parity=parity):
                buf, sem_i, sem_o = bank_of[parity]
                for b in range(NBUF):
                    s = (k2 - 2) * NBUF + b
                    @pl.when(jnp.logical_and(k2 >= 2, s < n_steps))
                    def _(b=b, s=s):
                        drain_one(sem_o)
                for b in range(NBUF):
                    s = k2 * NBUF + b
                    @pl.when(s < n_steps)
                    def _(b=b, s=s):
                        fire_in(s, buf, sem_i, b)
                pbuf, psem_i, psem_o = bank_of[1 - parity]
                for b in range(NBUF):
                    s = (k2 - 1) * NBUF + b
                    @pl.when(jnp.logical_and(k2 >= 1, s < n_steps))
                    def _(b=b, s=s):
                        drain_in_one(psem_i)
                for b in range(NBUF):
                    s = (k2 - 1) * NBUF + b
                    @pl.when(jnp.logical_and(k2 >= 1, s < n_steps))
                    def _(b=b, s=s):
                        fire_out(s, pbuf, psem_o, b)
        return _

    lax.fori_loop(0, n_outer_val + 3, outer, None)


def _gather_body(u_hbm, i_hbm, j_hbm, ulin_hbm, v_hbm, utail_hbm, out_hbm,
                 idx_u, idx_i, idx_j, widx, urows, irows, jrows, utail_v,
                 outv, sem_u, sem_v, *, b_per_w, nc):
    wid = lax.axis_index("s") * nc + lax.axis_index("c")
    base = wid * b_per_w
    nrow = b_per_w // 128
    row0 = wid * nrow

    pltpu.sync_copy(u_hbm.at[pl.ds(row0, nrow)], idx_u)
    pltpu.sync_copy(i_hbm.at[pl.ds(row0, nrow)], idx_i)
    pltpu.sync_copy(j_hbm.at[pl.ds(row0, nrow)], idx_j)
    pltpu.sync_copy(utail_hbm, utail_v)

    hv = []
    for c in range(nrow):
        sl = pl.ds(c * 128, 128)
        hv.append(pltpu.async_copy(v_hbm.at[idx_i.at[c]], irows.at[sl], sem_v))
        hv.append(pltpu.async_copy(v_hbm.at[idx_j.at[c]], jrows.at[sl], sem_v))

    lane_iota = lax.iota(jnp.int32, L)

    def widx_step(g, _):
        pos = g * L + lane_iota
        r = plsc.load_gather(idx_u, [pos >> 7, pos & 127])
        b0 = ((r >> 7) << 10) + (r & 127)
        for d in range(D):
            w = b0 + ((d // 8) * NTILE_U * 1024 + (d % 8) * 128)
            widx[d, pl.ds(g * L, L)] = w
        return _

    lax.fori_loop(0, b_per_w // L, widx_step, None)

    ngrp = b_per_w // 128
    hu = []
    for d in range(D):
        for c in range(ngrp):
            off = c * 128
            hu.append(pltpu.async_copy(
                ulin_hbm.at[widx.at[d, pl.ds(off, 128)]],
                urows.at[d, pl.ds(off, 128)], sem_u))
    for h in hv:
        h.wait()
    for h in hu:
        h.wait()

    def body(g, _):
        rows16 = g * L + lane_iota
        pos = g * L + lane_iota
        r = plsc.load_gather(idx_u, [pos >> 7, pos & 127])
        is_tail = r >= N_FULL * 128
        tr = jnp.maximum(r - N_FULL * 128, 0)
        acc = None
        for d in range(D):
            dcol = jnp.full((L,), d, jnp.int32)
            diff = (plsc.load_gather(irows, [rows16, dcol])
                    - plsc.load_gather(jrows, [rows16, dcol]))
            uv = urows[d, pl.ds(g * L, L)]
            tv = plsc.load_gather(utail_v, [tr, dcol])
            term = jnp.where(is_tail, tv, uv) * diff
            acc = term if acc is None else acc + term
        outv[pl.ds(g * L, L)] = 1.0 / (1.0 + jnp.exp(-acc))
        return _

    lax.fori_loop(0, b_per_w // L, body, None)

    pltpu.sync_copy(outv, out_hbm.at[pl.ds(base, b_per_w)])


def kernel(u, i, j, U, V):
    try:
        info = plsc.get_sparse_core_info()
        nc, ns = info.num_cores, info.num_subcores
    except ValueError:
        nc, ns = 2, 16
    nw = nc * ns
    b_per_w = BATCH // nw

    mesh = plsc.VectorSubcoreMesh(core_axis_name="c", subcore_axis_name="s")

    ut3 = U.T.reshape(4, 8, N_U)
    copy_k = functools.partial(
        pl.kernel,
        mesh=mesh,
        out_type=jax.ShapeDtypeStruct((4 * NTILE_U * 8, 128), jnp.float32),
        compiler_params=pltpu.CompilerParams(
            needs_layout_passes=False, use_tc_tiling_on_sc=True
        ),
        scratch_types=[
            pltpu.VMEM((8 * NBUF, 128 * KTR), jnp.float32),
            pltpu.VMEM((8 * NBUF, 128 * KTR), jnp.float32),
            pltpu.SemaphoreType.DMA,
            pltpu.SemaphoreType.DMA,
            pltpu.SemaphoreType.DMA,
            pltpu.SemaphoreType.DMA,
        ],
    )(functools.partial(_copy_body, nc=nc))
    ulin = copy_k(ut3).reshape(4 * NTILE_U * 8 * 128)

    u2 = u.astype(jnp.int32).reshape(BATCH // 128, 128)
    i2 = i.astype(jnp.int32).reshape(BATCH // 128, 128)
    j2 = j.astype(jnp.int32).reshape(BATCH // 128, 128)

    gather_k = functools.partial(
        pl.kernel,
        mesh=mesh,
        out_type=jax.ShapeDtypeStruct((BATCH,), jnp.float32),
        compiler_params=pltpu.CompilerParams(
            needs_layout_passes=False, use_tc_tiling_on_sc=False
        ),
        scratch_types=[
            pltpu.VMEM((BATCH // 128 // nw, 128), jnp.int32),
            pltpu.VMEM((BATCH // 128 // nw, 128), jnp.int32),
            pltpu.VMEM((BATCH // 128 // nw, 128), jnp.int32),
            pltpu.VMEM((D, b_per_w), jnp.int32),
            pltpu.VMEM((D, b_per_w), jnp.float32),
            pltpu.VMEM((b_per_w, D), jnp.float32),
            pltpu.VMEM((b_per_w, D), jnp.float32),
            pltpu.VMEM((N_U - N_FULL * 128, D), jnp.float32),
            pltpu.VMEM((b_per_w,), jnp.float32),
            pltpu.SemaphoreType.DMA,
            pltpu.SemaphoreType.DMA,
        ],
    )(functools.partial(_gather_body, b_per_w=b_per_w, nc=nc))
    utail = U[N_FULL * 128:, :]
    return gather_k(u2, i2, j2, ulin, V, utail)
